# baseline (speedup 1.0000x reference)
import jax
import jax.numpy as jnp
from jax import lax
from jax.experimental import pallas as pl
from jax.experimental.pallas import tpu as pltpu
from jax.experimental.pallas import tpu_sc as plsc

D_MODEL = 64
SEQ = 200
HSEQ = SEQ // 2
NC, NS = 2, 16
NW = NC * NS
LANES = 16
VPR = D_MODEL // LANES
SEQ_W = 128
PAIRS_W = SEQ_W // 2
EPI_BP = 16


def _emb_body(x_hbm, table_hbm, pos_hbm, out_hbm,
              idx_v, pos_v, gbuf, obuf,
              isem0, isem1, gsem0, gsem1, osem0, osem1):
    wid = lax.axis_index("s") * NC + lax.axis_index("c")

    pltpu.sync_copy(pos_hbm, pos_v)

    isems = (isem0, isem1)
    gsems = (gsem0, gsem1)
    osems = (osem0, osem1)

    def idx_copy(p, buf):
        return pltpu.make_async_copy(
            x_hbm.at[wid, pl.ds(4 * p, 4)], idx_v.at[buf], isems[buf])

    def gather_copies(buf):
        return tuple(
            pltpu.make_async_copy(
                table_hbm.at[idx_v.at[buf, k]],
                gbuf.at[buf, pl.ds(k * HSEQ, HSEQ)], gsems[buf])
            for k in range(4))

    def out_copy(p, buf):
        return pltpu.make_async_copy(
            obuf.at[buf], out_hbm.at[wid * PAIRS_W + p], osems[buf])

    idx_copy(0, 0).start()
    idx_copy(1, 1).start()
    for b in range(2):
        idx_copy(b, b).wait()
        for cp in gather_copies(b):
            cp.start()

    def chunk(t, b):
        p = 2 * t + b
        for cp in gather_copies(b):
            cp.wait()
        @pl.when(t > 0)
        def _():
            out_copy(p - 2, b).wait()
        @pl.when(p + 2 < PAIRS_W)
        def _():
            idx_copy(p + 2, b).start()

        def pack_rows(r, _):
            for u in range(2):
                s = 2 * r + u
                for j in range(VPR):
                    sl = pl.ds(j * LANES, LANES)
                    sh = pl.ds(D_MODEL + j * LANES, LANES)
                    pv = pos_v[s, sl]
                    obuf[b, s, sl] = gbuf[b, s, sl] + pv
                    obuf[b, s, sh] = gbuf[b, SEQ + s, sl] + pv
            return 0

        lax.fori_loop(0, SEQ // 2, pack_rows, 0)

        @pl.when(p + 2 < PAIRS_W)
        def _():
            idx_copy(p + 2, b).wait()
            for cp in gather_copies(b):
                cp.start()

        out_copy(p, b).start()

    def step(t, _):
        chunk(t, 0)
        chunk(t, 1)
        return 0

    lax.fori_loop(0, PAIRS_W // 2, step, 0)

    for b in range(2):
        out_copy(PAIRS_W - 2 + b, b).wait()


def _unpack_body(pk_ref, out_ref):
    for q in range(EPI_BP):
        blk = pk_ref[q]
        out_ref[2 * q] = blk[:, :D_MODEL]
        out_ref[2 * q + 1] = blk[:, D_MODEL:]


def kernel(x, token_table, pos_table):
    B, S = x.shape
    total = B * S
    x3 = x.astype(jnp.int32).reshape(NW, total // (NW * HSEQ), HSEQ)

    mesh = plsc.VectorSubcoreMesh(core_axis_name="c", subcore_axis_name="s")
    packed = pl.kernel(
        _emb_body,
        out_type=jax.ShapeDtypeStruct((B // 2, SEQ, 2 * D_MODEL),
                                      jnp.float32),
        mesh=mesh,
        compiler_params=pltpu.CompilerParams(use_tc_tiling_on_sc=False),
        scratch_types=[
            pltpu.VMEM((2, 4, HSEQ), jnp.int32),
            pltpu.VMEM((SEQ, D_MODEL), jnp.float32),
            pltpu.VMEM((2, 2 * SEQ, D_MODEL), jnp.float32),
            pltpu.VMEM((2, SEQ, 2 * D_MODEL), jnp.float32),
            pltpu.SemaphoreType.DMA,
            pltpu.SemaphoreType.DMA,
            pltpu.SemaphoreType.DMA,
            pltpu.SemaphoreType.DMA,
            pltpu.SemaphoreType.DMA,
            pltpu.SemaphoreType.DMA,
        ],
    )(x3, token_table, pos_table)

    grid = B // (2 * EPI_BP)
    return pl.pallas_call(
        _unpack_body,
        grid=(grid,),
        in_specs=[pl.BlockSpec((EPI_BP, SEQ, 2 * D_MODEL),
                               lambda i: (i, 0, 0))],
        out_specs=pl.BlockSpec((2 * EPI_BP, SEQ, D_MODEL),
                               lambda i: (i, 0, 0)),
        out_shape=jax.ShapeDtypeStruct((B, SEQ, D_MODEL), jnp.float32),
        compiler_params=pltpu.CompilerParams(
            dimension_semantics=("arbitrary",)),
    )(packed)
